# baseline (speedup 1.0000x reference)
import functools

import jax
import jax.numpy as jnp
from jax import lax
from jax.experimental import pallas as pl
from jax.experimental.pallas import tpu as pltpu
from jax.experimental.pallas import tpu_sc as plsc

N_FACTORS = 32
BATCH = 16384
PACK = 128 // N_FACTORS

NC = 2
NS = 16
LANES = 16
NW = NC * NS
BPW = BATCH // NW
CHUNK = 128
NCHUNK = BPW // CHUNK
GSIZE = CHUNK // LANES


def _mf_kernel(u_hbm, m_hbm, U_hbm, M_hbm, out_hbm,
               idx_u, idx_m, gidx_u, gidx_m, rows_u, rows_m, out_v,
               sem0, sem1):
    wid = lax.axis_index("s") * NC + lax.axis_index("c")
    base = wid * BPW

    pltpu.sync_copy(u_hbm.at[pl.ds(base, BPW)], idx_u)
    pltpu.sync_copy(m_hbm.at[pl.ds(base, BPW)], idx_m)

    def shift_body(t, carry):
        s = pl.ds(t * LANES, LANES)
        gidx_u[s] = idx_u[s] // PACK
        gidx_m[s] = idx_m[s] // PACK
        return carry

    lax.fori_loop(0, BPW // LANES, shift_body, 0)

    sems = (sem0, sem1)

    def fire(j):
        p = j % 2
        s = pl.ds(j * CHUNK, CHUNK)
        return (
            pltpu.async_copy(U_hbm.at[gidx_u.at[s]], rows_u[p], sems[p]),
            pltpu.async_copy(M_hbm.at[gidx_m.at[s]], rows_m[p], sems[p]),
        )

    lanes = lax.iota(jnp.int32, LANES)
    inflight = fire(0)
    for j in range(NCHUNK):
        nxt = fire(j + 1) if j + 1 < NCHUNK else None
        for c in inflight:
            c.wait()
        p = j % 2

        def group_body(g, carry, p=p, j=j):
            row = jnp.full((LANES,), g * LANES, jnp.int32) + lanes
            s = pl.ds(j * CHUNK + g * LANES, LANES)
            off_u = (idx_u[s] % PACK) * N_FACTORS
            off_m = (idx_m[s] % PACK) * N_FACTORS
            acc = jnp.zeros((LANES,), jnp.float32)
            for d in range(N_FACTORS):
                uv = plsc.load_gather(rows_u[p], [row, off_u + d])
                mv = plsc.load_gather(rows_m[p], [row, off_m + d])
                acc = acc + uv * mv
            out_v[s] = acc
            return carry

        lax.fori_loop(0, GSIZE, group_body, 0)
        inflight = nxt

    pltpu.sync_copy(out_v, out_hbm.at[pl.ds(base, BPW)])


def kernel(users, movies, U, M):
    users = users.astype(jnp.int32)
    movies = movies.astype(jnp.int32)
    U128 = U.reshape(U.shape[0] // PACK, 128)
    M128 = M.reshape(M.shape[0] // PACK, 128)

    mesh = plsc.VectorSubcoreMesh(core_axis_name="c", subcore_axis_name="s")
    k = functools.partial(
        pl.kernel,
        mesh=mesh,
        compiler_params=pltpu.CompilerParams(needs_layout_passes=False),
        out_type=jax.ShapeDtypeStruct((BATCH,), jnp.float32),
        scratch_types=[
            pltpu.VMEM((BPW,), jnp.int32),
            pltpu.VMEM((BPW,), jnp.int32),
            pltpu.VMEM((BPW,), jnp.int32),
            pltpu.VMEM((BPW,), jnp.int32),
            [pltpu.VMEM((CHUNK, 128), jnp.float32)] * 2,
            [pltpu.VMEM((CHUNK, 128), jnp.float32)] * 2,
            pltpu.VMEM((BPW,), jnp.float32),
            pltpu.SemaphoreType.DMA,
            pltpu.SemaphoreType.DMA,
        ],
    )(_mf_kernel)
    return k(users, movies, U128, M128)


# speedup vs baseline: 3.6100x; 3.6100x over previous
import functools

import jax
import jax.numpy as jnp
from jax import lax
from jax.experimental import pallas as pl
from jax.experimental.pallas import tpu as pltpu
from jax.experimental.pallas import tpu_sc as plsc

F = 32
BATCH = 16384
NU = 1000000
NM = 100000
NC = 2
NS = 16
LANES = 16
FPC = F // NC
EPT = BATCH // NS
GROUPS = EPT // LANES

HALF = 499968
UMAIN = 2 * HALF
NTAIL = NU - UMAIN
SENT = 128
UBUF = UMAIN + SENT
SEG_LENS = (124928, 124928, 124928, 124928, 256)
SEG_OFFS = (0, 124928, 249856, 374784, 499712)
STAGER0 = 10

SLOTS = []
for _f in range(FPC):
    SLOTS.append(("M", _f))
    SLOTS.append(("U", _f, 0))
    SLOTS.append(("U", _f, 1))
NSLOT = len(SLOTS)


def _mf_main(u_hbm, m_hbm, UT_hbm, MT_hbm, utail_hbm, part_hbm,
             usr_v, midx_v, uidx0, uidx1, uv0, uv1, uv2, uv3,
             mt0, mt1, acc_v, tbase_v, tmask_v, utail_v, zb_v,
             u_sh, m_sh0, m_sh1,
             stage_sem, gather_sem):
    c = lax.axis_index("c")
    s = lax.axis_index("s")
    lanes = lax.iota(jnp.int32, LANES)
    uvs = (uv0, uv1, uv2, uv3)
    uidxs = (uidx0, uidx1)
    mts = (mt0, mt1)
    m_shs = (m_sh0, m_sh1)

    pltpu.sync_copy(u_hbm.at[pl.ds(s * EPT, EPT)], usr_v)
    pltpu.sync_copy(m_hbm.at[pl.ds(s * EPT, EPT)], midx_v)
    pltpu.sync_copy(utail_hbm, utail_v)

    @pl.when(s == NS - 1)
    def _zero_sentinels():
        zero = jnp.zeros((LANES,), jnp.float32)
        for g in range(SENT // LANES):
            zb_v[pl.ds(g * LANES, LANES)] = zero
        pltpu.sync_copy(zb_v, u_sh.at[pl.ds(UMAIN, SENT)])

    base_col = c * FPC

    def prep_body(g, ht):
        sl = pl.ds(g * LANES, LANES)
        u = usr_v[sl]
        sent = UMAIN + (g % 8) * LANES + lanes
        uidx0[sl] = jnp.where(u < HALF, u, sent)
        uidx1[sl] = jnp.where(
            jnp.logical_and(u >= HALF, u < UMAIN), u, sent)
        acc_v[sl] = jnp.zeros((LANES,), jnp.float32)
        istail = u >= UMAIN
        tmask_v[sl] = jnp.where(istail, 1.0, 0.0).astype(jnp.float32)
        tbase_v[sl] = jnp.where(istail, (u - UMAIN) * F + base_col, 0)
        anyt = lax.reduce_max(
            jnp.where(istail, 1, 0).astype(jnp.int32), (0,))
        return jnp.maximum(ht, anyt)

    has_tail = lax.fori_loop(0, GROUPS, prep_body, jnp.int32(0))

    def issue_stage(j):
        slot = SLOTS[j]
        if slot[0] == "M":
            f = slot[1]
            @pl.when(s == NS - 1)
            def _(f=f):
                pltpu.async_copy(
                    MT_hbm.at[c * FPC + f], m_shs[f % 2], stage_sem)
        else:
            f, h = slot[1], slot[2]
            d = c * FPC + f
            for i, (off, ln) in enumerate(zip(SEG_OFFS, SEG_LENS)):
                @pl.when(s == STAGER0 + i)
                def _(d=d, h=h, off=off, ln=ln):
                    pltpu.async_copy(
                        UT_hbm.at[d, pl.ds(h * HALF + off, ln)],
                        u_sh.at[pl.ds(h * HALF + off, ln)], stage_sem)

    def drain_stage(j):
        slot = SLOTS[j]
        if slot[0] == "M":
            @pl.when(s == NS - 1)
            def _():
                pltpu.make_async_copy(
                    MT_hbm.at[0], m_sh0, stage_sem).wait()
        else:
            for i, ln in enumerate(SEG_LENS):
                @pl.when(s == STAGER0 + i)
                def _(ln=ln):
                    pltpu.make_async_copy(
                        UT_hbm.at[0, pl.ds(0, ln)],
                        u_sh.at[pl.ds(0, ln)], stage_sem).wait()

    def fire_gather(j):
        slot = SLOTS[j]
        if slot[0] == "M":
            f = slot[1]
            return pltpu.async_copy(
                m_shs[f % 2].at[midx_v], mts[f % 2], gather_sem)
        f, h = slot[1], slot[2]
        return pltpu.async_copy(
            u_sh.at[uidxs[h]], uvs[(2 * f + h) % 4], gather_sem)

    def fma(f):
        r0 = (2 * f) % 4
        r1 = (2 * f + 1) % 4
        mt = mts[f % 2]

        @pl.when(has_tail == 0)
        def _plain():
            def body(g, carry):
                sl = pl.ds(g * LANES, LANES)
                uval = uvs[r0][sl] + uvs[r1][sl]
                acc_v[sl] = acc_v[sl] + uval * mt[sl]
                return carry

            lax.fori_loop(0, GROUPS, body, 0)

        @pl.when(has_tail == 1)
        def _with_tail():
            def body(g, carry, f=f):
                sl = pl.ds(g * LANES, LANES)
                tv = plsc.load_gather(utail_v, [tbase_v[sl] + f])
                uval = uvs[r0][sl] + uvs[r1][sl] + tv * tmask_v[sl]
                acc_v[sl] = acc_v[sl] + uval * mt[sl]
                return carry

            lax.fori_loop(0, GROUPS, body, 0)

    issue_stage(0)
    inflight = None
    for j in range(NSLOT):
        if inflight is not None:
            inflight.wait()
        drain_stage(j)
        plsc.subcore_barrier()
        if j + 1 < NSLOT:
            issue_stage(j + 1)
        inflight = fire_gather(j)
        if j >= 4 and (j - 4) % 3 == 0:
            fma((j - 4) // 3)
    inflight.wait()
    fma(FPC - 1)

    pltpu.sync_copy(acc_v, part_hbm.at[c, pl.ds(s * EPT, EPT)])


def _add_kernel(part_hbm, out_hbm, a_v, b_v):
    wid = lax.axis_index("s") * NC + lax.axis_index("c")
    n = BATCH // (NC * NS)
    base = wid * n
    pltpu.sync_copy(part_hbm.at[0, pl.ds(base, n)], a_v)
    pltpu.sync_copy(part_hbm.at[1, pl.ds(base, n)], b_v)

    def body(g, carry):
        sl = pl.ds(g * LANES, LANES)
        a_v[sl] = a_v[sl] + b_v[sl]
        return carry

    lax.fori_loop(0, n // LANES, body, 0)
    pltpu.sync_copy(a_v, out_hbm.at[pl.ds(base, n)])


def kernel(users, movies, U, M):
    users = users.astype(jnp.int32)
    movies = movies.astype(jnp.int32)
    UT = U.T
    MT = M.T
    utail = U[UMAIN:].reshape(-1)

    mesh = plsc.VectorSubcoreMesh(core_axis_name="c", subcore_axis_name="s")
    params = pltpu.CompilerParams(needs_layout_passes=False)

    k1 = functools.partial(
        pl.kernel,
        mesh=mesh,
        compiler_params=params,
        out_type=jax.ShapeDtypeStruct((NC, BATCH), jnp.float32),
        scratch_types=[
            pltpu.VMEM((EPT,), jnp.int32),
            pltpu.VMEM((EPT,), jnp.int32),
            pltpu.VMEM((EPT,), jnp.int32),
            pltpu.VMEM((EPT,), jnp.int32),
            pltpu.VMEM((EPT,), jnp.float32),
            pltpu.VMEM((EPT,), jnp.float32),
            pltpu.VMEM((EPT,), jnp.float32),
            pltpu.VMEM((EPT,), jnp.float32),
            pltpu.VMEM((EPT,), jnp.float32),
            pltpu.VMEM((EPT,), jnp.float32),
            pltpu.VMEM((EPT,), jnp.float32),
            pltpu.VMEM((EPT,), jnp.int32),
            pltpu.VMEM((EPT,), jnp.float32),
            pltpu.VMEM((NTAIL * F,), jnp.float32),
            pltpu.VMEM((SENT,), jnp.float32),
            pltpu.VMEM_SHARED((UBUF,), jnp.float32),
            pltpu.VMEM_SHARED((NM,), jnp.float32),
            pltpu.VMEM_SHARED((NM,), jnp.float32),
            pltpu.SemaphoreType.DMA,
            pltpu.SemaphoreType.DMA,
        ],
    )(_mf_main)
    partials = k1(users, movies, UT, MT, utail)

    k2 = functools.partial(
        pl.kernel,
        mesh=mesh,
        compiler_params=params,
        out_type=jax.ShapeDtypeStruct((BATCH,), jnp.float32),
        scratch_types=[
            pltpu.VMEM((BATCH // (NC * NS),), jnp.float32),
            pltpu.VMEM((BATCH // (NC * NS),), jnp.float32),
        ],
    )(_add_kernel)
    return k2(partials)


# speedup vs baseline: 4.0277x; 1.1157x over previous
import functools

import jax
import jax.numpy as jnp
from jax import lax
from jax.experimental import pallas as pl
from jax.experimental.pallas import tpu as pltpu
from jax.experimental.pallas import tpu_sc as plsc

F = 32
BATCH = 16384
NU = 1000000
NM = 100000
NC = 2
NS = 16
LANES = 16
FPC = F // NC
EPT = BATCH // NS
GROUPS = EPT // LANES

HALF = 499968
UMAIN = 2 * HALF
NTAIL = NU - UMAIN
SENT = 128
UBUF = UMAIN + SENT
SEG_A = 33280
SEG_B = 34048
NSLOT = 3 * FPC


def _mf_main(u_hbm, m_hbm, UT_hbm, MT_hbm, utail_hbm, part_hbm,
             usr_v, midx_v, uidx_v, uv0, uv1, mt0, mt1,
             acc_v, tbase_v, tmask_v, utail_v, zb_v,
             u_sh, m_sh0, m_sh1,
             stage_sem, mstage_sem, gather_sem):
    c = lax.axis_index("c")
    s = lax.axis_index("s")
    lanes = lax.iota(jnp.int32, LANES)
    uvs = (uv0, uv1)
    mts = (mt0, mt1)
    m_shs = (m_sh0, m_sh1)

    pltpu.sync_copy(u_hbm.at[pl.ds(s * EPT, EPT)], usr_v)
    pltpu.sync_copy(m_hbm.at[pl.ds(s * EPT, EPT)], midx_v)
    pltpu.sync_copy(utail_hbm, utail_v)

    @pl.when(s == NS - 1)
    def _zero_sentinels():
        zero = jnp.zeros((LANES,), jnp.float32)
        for g in range(SENT // LANES):
            zb_v[pl.ds(g * LANES, LANES)] = zero
        pltpu.sync_copy(zb_v, u_sh.at[pl.ds(UMAIN, SENT)])

    base_col = c * FPC

    def prep_body(g, ht):
        sl = pl.ds(g * LANES, LANES)
        u = usr_v[sl]
        sent = UMAIN + (g % 8) * LANES + lanes
        uidx_v[sl] = jnp.where(u < UMAIN, u, sent)
        acc_v[sl] = jnp.zeros((LANES,), jnp.float32)
        istail = u >= UMAIN
        tmask_v[sl] = jnp.where(istail, 1.0, 0.0).astype(jnp.float32)
        tbase_v[sl] = jnp.where(istail, (u - UMAIN) * F + base_col, 0)
        anyt = lax.reduce_max(
            jnp.where(istail, 1, 0).astype(jnp.int32), (0,))
        return jnp.maximum(ht, anyt)

    has_tail = lax.fori_loop(0, GROUPS, prep_body, jnp.int32(0))

    def issue_u_stage(f, h):
        d = c * FPC + f

        @pl.when(s < NS - 2)
        def _():
            off = pl.multiple_of(h * HALF + s * SEG_A, 128)
            pltpu.async_copy(
                UT_hbm.at[d, pl.ds(off, SEG_A)],
                u_sh.at[pl.ds(off, SEG_A)], stage_sem)

        @pl.when(s == NS - 2)
        def _():
            off = h * HALF + 14 * SEG_A
            pltpu.async_copy(
                UT_hbm.at[d, pl.ds(off, SEG_B)],
                u_sh.at[pl.ds(off, SEG_B)], stage_sem)

    def drain_u_stage():
        @pl.when(s < NS - 2)
        def _():
            pltpu.make_async_copy(
                UT_hbm.at[0, pl.ds(0, SEG_A)],
                u_sh.at[pl.ds(0, SEG_A)], stage_sem).wait()

        @pl.when(s == NS - 2)
        def _():
            pltpu.make_async_copy(
                UT_hbm.at[0, pl.ds(0, SEG_B)],
                u_sh.at[pl.ds(0, SEG_B)], stage_sem).wait()

    def issue_m_stage(f):
        @pl.when(s == NS - 1)
        def _():
            pltpu.async_copy(
                MT_hbm.at[c * FPC + f], m_shs[f % 2], mstage_sem)

    def drain_m_stage():
        @pl.when(s == NS - 1)
        def _():
            pltpu.make_async_copy(MT_hbm.at[0], m_sh0, mstage_sem).wait()

    def fma(f):
        uv = uvs[f % 2]
        mt = mts[f % 2]

        @pl.when(has_tail == 0)
        def _plain():
            def body(g, carry):
                sl = pl.ds(g * LANES, LANES)
                acc_v[sl] = acc_v[sl] + uv[sl] * mt[sl]
                return carry

            lax.fori_loop(0, GROUPS, body, 0)

        @pl.when(has_tail == 1)
        def _with_tail():
            def body(g, carry, f=f):
                sl = pl.ds(g * LANES, LANES)
                tv = plsc.load_gather(utail_v, [tbase_v[sl] + f])
                uval = uv[sl] + tv * tmask_v[sl]
                acc_v[sl] = acc_v[sl] + uval * mt[sl]
                return carry

            lax.fori_loop(0, GROUPS, body, 0)

    issue_m_stage(0)
    inflight = None
    for j in range(NSLOT):
        f, sub = divmod(j, 3)
        if inflight is not None:
            inflight.wait()
            inflight = None
        if sub == 0:
            drain_m_stage()
        elif sub == 2:
            drain_u_stage()
            drain_u_stage()
        plsc.subcore_barrier()
        if sub == 0:
            if f + 1 < FPC:
                issue_m_stage(f + 1)
            issue_u_stage(f, 0)
            issue_u_stage(f, 1)
            inflight = pltpu.async_copy(
                m_shs[f % 2].at[midx_v], mts[f % 2], gather_sem)
        elif sub == 2:
            inflight = pltpu.async_copy(
                u_sh.at[uidx_v], uvs[f % 2], gather_sem)
        if j >= 4 and (j - 4) % 3 == 0:
            fma((j - 4) // 3)
    inflight.wait()
    fma(FPC - 1)

    pltpu.sync_copy(acc_v, part_hbm.at[c, pl.ds(s * EPT, EPT)])


def _add_kernel(part_hbm, out_hbm, a_v, b_v):
    wid = lax.axis_index("s") * NC + lax.axis_index("c")
    n = BATCH // (NC * NS)
    base = wid * n
    pltpu.sync_copy(part_hbm.at[0, pl.ds(base, n)], a_v)
    pltpu.sync_copy(part_hbm.at[1, pl.ds(base, n)], b_v)

    def body(g, carry):
        sl = pl.ds(g * LANES, LANES)
        a_v[sl] = a_v[sl] + b_v[sl]
        return carry

    lax.fori_loop(0, n // LANES, body, 0)
    pltpu.sync_copy(a_v, out_hbm.at[pl.ds(base, n)])


def kernel(users, movies, U, M):
    users = users.astype(jnp.int32)
    movies = movies.astype(jnp.int32)
    UT = U.T
    MT = M.T
    utail = U[UMAIN:].reshape(-1)

    mesh = plsc.VectorSubcoreMesh(core_axis_name="c", subcore_axis_name="s")
    params = pltpu.CompilerParams(needs_layout_passes=False)

    k1 = functools.partial(
        pl.kernel,
        mesh=mesh,
        compiler_params=params,
        out_type=jax.ShapeDtypeStruct((NC, BATCH), jnp.float32),
        scratch_types=[
            pltpu.VMEM((EPT,), jnp.int32),
            pltpu.VMEM((EPT,), jnp.int32),
            pltpu.VMEM((EPT,), jnp.int32),
            pltpu.VMEM((EPT,), jnp.float32),
            pltpu.VMEM((EPT,), jnp.float32),
            pltpu.VMEM((EPT,), jnp.float32),
            pltpu.VMEM((EPT,), jnp.float32),
            pltpu.VMEM((EPT,), jnp.float32),
            pltpu.VMEM((EPT,), jnp.int32),
            pltpu.VMEM((EPT,), jnp.float32),
            pltpu.VMEM((NTAIL * F,), jnp.float32),
            pltpu.VMEM((SENT,), jnp.float32),
            pltpu.VMEM_SHARED((UBUF,), jnp.float32),
            pltpu.VMEM_SHARED((NM,), jnp.float32),
            pltpu.VMEM_SHARED((NM,), jnp.float32),
            pltpu.SemaphoreType.DMA,
            pltpu.SemaphoreType.DMA,
            pltpu.SemaphoreType.DMA,
        ],
    )(_mf_main)
    partials = k1(users, movies, UT, MT, utail)

    k2 = functools.partial(
        pl.kernel,
        mesh=mesh,
        compiler_params=params,
        out_type=jax.ShapeDtypeStruct((BATCH,), jnp.float32),
        scratch_types=[
            pltpu.VMEM((BATCH // (NC * NS),), jnp.float32),
            pltpu.VMEM((BATCH // (NC * NS),), jnp.float32),
        ],
    )(_add_kernel)
    return k2(partials)
